# baseline (speedup 1.0000x reference)
import functools

import jax
import jax.numpy as jnp
from jax import lax
from jax.experimental import pallas as pl
from jax.experimental.pallas import tpu as pltpu
from jax.experimental.pallas import tpu_sc as plsc

VOCAB = 1000000
DIM = 300
BATCH = 16384
MAX_NORM = 1.0

_NC = 2
_NS = 16
_NW = _NC * _NS
_BPW = BATCH // _NW
_L = 16
_DPAD = 384
_NG = (DIM + _L - 1) // _L
_RSTAGE = 16
_VB = 128
_VBMAX = VOCAB - _VB
_BSTRIDE = 304


def _sc_gather2(ci_sorted, ci_pos, xi_sorted, xi_pos, Wc_t, Wx_t):
    mesh = plsc.VectorSubcoreMesh(core_axis_name="c", subcore_axis_name="s")

    @functools.partial(
        pl.kernel,
        out_type=(
            jax.ShapeDtypeStruct((BATCH, _DPAD), jnp.float32),
            jax.ShapeDtypeStruct((BATCH, _DPAD), jnp.float32),
        ),
        mesh=mesh,
        compiler_params=pltpu.CompilerParams(
            use_tc_tiling_on_sc=True, needs_layout_passes=False),
        scratch_types=[
            pltpu.VMEM((_BPW + _VB + _L,), jnp.int32),
            pltpu.VMEM((_BPW,), jnp.int32),
            pltpu.VMEM((_BPW + _L,), jnp.int32),
            pltpu.VMEM((2 * _BSTRIDE, _VB), jnp.float32),
            pltpu.VMEM((2 * _RSTAGE, _DPAD), jnp.float32),
            pltpu.VMEM((_BPW // _RSTAGE, _L), jnp.int32),
            pltpu.SemaphoreType.DMA,
            pltpu.SemaphoreType.DMA,
        ],
    )
    def k(ci_hbm, cp_hbm, xi_hbm, xp_hbm, wc_hbm, wx_hbm, outc_hbm, outx_hbm,
          idx_v, pos_v, dlist_v, blk_v, rows_v, spos_v, sem, osem):
        wid = lax.axis_index("s") * _NC + lax.axis_index("c")

        lanes = lax.iota(jnp.int32, _L)

        def blocks_of(vec):
            return jnp.minimum((vec // _VB) * _VB, _VBMAX)

        def gather_one(idx_hbm, pos_hbm, wt_hbm, out_hbm):
            pltpu.sync_copy(idx_hbm.at[wid], idx_v.at[pl.ds(_VB, _BPW)])
            pltpu.sync_copy(pos_hbm.at[wid], pos_v)

            def scan_body(kc, nd):
                a = blocks_of(idx_v[pl.ds(_VB + kc * _L, _L)])
                b = blocks_of(idx_v[pl.ds(_VB - 1 + kc * _L, _L)])
                flags = (a != b) | ((lanes + kc) == 0)
                plsc.store_compressed(dlist_v.at[pl.ds(nd, _L)], a, mask=flags)
                return nd + plsc.all_reduce_population_count(flags)[0]

            nd = lax.fori_loop(0, _BPW // _L, scan_body, jnp.int32(0))

            def issue_fetch(b):
                vb = pl.multiple_of(dlist_v[pl.ds(b, _L)][0], _VB)
                off = pl.multiple_of((b % 2) * _BSTRIDE, 8)
                pltpu.async_copy(
                    wt_hbm.at[:, pl.ds(vb, _VB)],
                    blk_v.at[pl.ds(off, DIM)], sem)

            def wait_fetch():
                pltpu.make_async_copy(
                    wt_hbm.at[:, pl.ds(0, _VB)],
                    blk_v.at[pl.ds(0, DIM)], sem).wait()

            def rows_half(f):
                h = pl.multiple_of((f % 2) * _RSTAGE, _RSTAGE)
                return rows_v.at[pl.ds(h, _RSTAGE)]

            def wait_flush(f):
                pltpu.make_async_copy(
                    rows_half(f), out_hbm.at[spos_v.at[f]], osem).wait()

            issue_fetch(jnp.int32(0))

            def body(j, carry):
                vb_cur, ordi = carry
                v = idx_v[pl.ds(_VB + j, _L)][0]
                vb = jnp.minimum((v // _VB) * _VB, _VBMAX)
                trans = vb != vb_cur
                ordn = jnp.where(trans, ordi + 1, ordi)

                @pl.when(trans)
                def _():
                    @pl.when(ordn + 1 < nd)
                    def _():
                        issue_fetch(ordn + 1)

                    wait_fetch()

                @pl.when((j % _RSTAGE == 0) & (j >= 2 * _RSTAGE))
                def _():
                    wait_flush(j // _RSTAGE - 2)

                off = (ordn % 2) * _BSTRIDE
                lane_idx = jnp.full((_L,), v - vb, jnp.int32)
                r = j % (2 * _RSTAGE)
                for g in range(_NG):
                    row_idx = jnp.minimum(lanes + (g * _L), DIM - 1) + off
                    rows_v[r, pl.ds(g * _L, _L)] = plsc.load_gather(
                        blk_v, [row_idx, lane_idx])

                @pl.when(j % _RSTAGE == _RSTAGE - 1)
                def _():
                    f = j // _RSTAGE
                    j0 = pl.multiple_of(j - (_RSTAGE - 1), _RSTAGE)
                    spos_v[f, :] = pos_v[pl.ds(j0, _L)]
                    pltpu.async_copy(
                        rows_half(f), out_hbm.at[spos_v.at[f]], osem)

                return vb, ordn

            lax.fori_loop(0, _BPW, body, (jnp.int32(-1), jnp.int32(-1)))
            wait_flush(jnp.int32(_BPW // _RSTAGE - 2))
            wait_flush(jnp.int32(_BPW // _RSTAGE - 1))

        gather_one(ci_hbm, cp_hbm, wc_hbm, outc_hbm)
        gather_one(xi_hbm, xp_hbm, wx_hbm, outx_hbm)

    return k(ci_sorted.reshape(_NW, _BPW), ci_pos.reshape(_NW, _BPW),
             xi_sorted.reshape(_NW, _BPW), xi_pos.reshape(_NW, _BPW),
             Wc_t, Wx_t)


_FBLK = 2048


def _tc_finish_body(c_ref, x_ref, out_ref):
    d = lax.broadcasted_iota(jnp.int32, (1, _DPAD), 1)
    mask = (d < DIM).astype(jnp.float32)
    c = c_ref[...] * mask
    x = x_ref[...] * mask
    dot = jnp.sum(c * x, axis=1)
    c2 = jnp.sum(c * c, axis=1)
    x2 = jnp.sum(x * x, axis=1)
    scale_c = jnp.minimum(1.0, MAX_NORM / jnp.maximum(jnp.sqrt(c2), 1e-7))
    scale_x = jnp.minimum(1.0, MAX_NORM / jnp.maximum(jnp.sqrt(x2), 1e-7))
    s = dot * scale_c * scale_x
    loss = jax.nn.log_sigmoid(s)
    part = jnp.full((1, 1), -jnp.sum(loss) / BATCH, jnp.float32)

    @pl.when(pl.program_id(0) == 0)
    def _():
        out_ref[...] = jnp.zeros((1, 1), jnp.float32)

    out_ref[...] += part


def kernel(center_input, context_input, W_center, W_context):
    ci = center_input.astype(jnp.int32)
    xi = context_input.astype(jnp.int32)
    iota = lax.iota(jnp.int32, BATCH)
    ci_s, ci_pos = lax.sort((ci, iota), num_keys=1)
    xi_s, xi_pos = lax.sort((xi, iota), num_keys=1)
    rows_c, rows_x = _sc_gather2(
        ci_s, ci_pos, xi_s, xi_pos, W_center.T, W_context.T)
    res = pl.pallas_call(
        _tc_finish_body,
        grid=(BATCH // _FBLK,),
        in_specs=[
            pl.BlockSpec((_FBLK, _DPAD), lambda i: (i, 0)),
            pl.BlockSpec((_FBLK, _DPAD), lambda i: (i, 0)),
        ],
        out_specs=pl.BlockSpec((1, 1), lambda i: (0, 0)),
        out_shape=jax.ShapeDtypeStruct((1, 1), jnp.float32),
    )(rows_c, rows_x)
    return res[0, 0]


# speedup vs baseline: 1.0210x; 1.0210x over previous
import functools

import jax
import jax.numpy as jnp
from jax import lax
from jax.experimental import pallas as pl
from jax.experimental.pallas import tpu as pltpu
from jax.experimental.pallas import tpu_sc as plsc

VOCAB = 1000000
DIM = 300
BATCH = 16384
MAX_NORM = 1.0

_NC = 2
_NS = 16
_NW = _NC * _NS
_BPW = BATCH // _NW
_L = 16
_DPAD = 384
_NG = (DIM + _L - 1) // _L
_RSTAGE = 16
_VB = 128
_VBMAX = VOCAB - _VB
_BSTRIDE = 304


def _sc_gather2(ci_sorted, ci_pos, xi_sorted, xi_pos, Wc_t, Wx_t):
    mesh = plsc.VectorSubcoreMesh(core_axis_name="c", subcore_axis_name="s")

    @functools.partial(
        pl.kernel,
        out_type=(
            jax.ShapeDtypeStruct((BATCH, _DPAD), jnp.float32),
            jax.ShapeDtypeStruct((BATCH, _DPAD), jnp.float32),
        ),
        mesh=mesh,
        compiler_params=pltpu.CompilerParams(
            use_tc_tiling_on_sc=True, needs_layout_passes=False),
        scratch_types=[
            pltpu.VMEM((_BPW + _VB + _L,), jnp.int32),
            pltpu.VMEM((_BPW,), jnp.int32),
            pltpu.VMEM((_BPW + _L,), jnp.int32),
            pltpu.VMEM((3 * _BSTRIDE, _VB), jnp.float32),
            pltpu.VMEM((_RSTAGE, _DPAD), jnp.float32),
            pltpu.VMEM((_BPW // _RSTAGE, _L), jnp.int32),
            pltpu.SemaphoreType.DMA,
            pltpu.SemaphoreType.DMA,
        ],
    )
    def k(ci_hbm, cp_hbm, xi_hbm, xp_hbm, wc_hbm, wx_hbm, outc_hbm, outx_hbm,
          idx_v, pos_v, dlist_v, blk_v, rows_v, spos_v, sem, osem):
        wid = lax.axis_index("s") * _NC + lax.axis_index("c")

        lanes = lax.iota(jnp.int32, _L)

        def blocks_of(vec):
            return jnp.minimum((vec // _VB) * _VB, _VBMAX)

        def gather_one(idx_hbm, pos_hbm, wt_hbm, out_hbm):
            pltpu.sync_copy(idx_hbm.at[wid], idx_v.at[pl.ds(_VB, _BPW)])
            pltpu.sync_copy(pos_hbm.at[wid], pos_v)

            def scan_body(kc, nd):
                a = blocks_of(idx_v[pl.ds(_VB + kc * _L, _L)])
                b = blocks_of(idx_v[pl.ds(_VB - 1 + kc * _L, _L)])
                flags = (a != b) | ((lanes + kc) == 0)
                plsc.store_compressed(dlist_v.at[pl.ds(nd, _L)], a, mask=flags)
                return nd + plsc.all_reduce_population_count(flags)[0]

            nd = lax.fori_loop(0, _BPW // _L, scan_body, jnp.int32(0))

            def issue_fetch(b):
                vb = pl.multiple_of(dlist_v[pl.ds(b, _L)][0], _VB)
                off = pl.multiple_of((b % 3) * _BSTRIDE, 8)
                pltpu.async_copy(
                    wt_hbm.at[:, pl.ds(vb, _VB)],
                    blk_v.at[pl.ds(off, DIM)], sem)

            def wait_fetch():
                pltpu.make_async_copy(
                    wt_hbm.at[:, pl.ds(0, _VB)],
                    blk_v.at[pl.ds(0, DIM)], sem).wait()

            issue_fetch(jnp.int32(0))

            @pl.when(nd > 1)
            def _():
                issue_fetch(jnp.int32(1))

            def body(j, carry):
                vb_cur, ordi = carry
                v = idx_v[pl.ds(_VB + j, _L)][0]
                vb = jnp.minimum((v // _VB) * _VB, _VBMAX)
                trans = vb != vb_cur
                ordn = jnp.where(trans, ordi + 1, ordi)

                @pl.when(trans)
                def _():
                    @pl.when(ordn + 2 < nd)
                    def _():
                        issue_fetch(ordn + 2)

                    wait_fetch()

                off = (ordn % 3) * _BSTRIDE
                lane_idx = jnp.full((_L,), v - vb, jnp.int32)
                r = j % _RSTAGE
                for g in range(_NG):
                    row_idx = jnp.minimum(lanes + (g * _L), DIM - 1) + off
                    rows_v[r, pl.ds(g * _L, _L)] = plsc.load_gather(
                        blk_v, [row_idx, lane_idx])

                @pl.when(r == _RSTAGE - 1)
                def _():
                    f = j // _RSTAGE
                    j0 = pl.multiple_of(j - (_RSTAGE - 1), _RSTAGE)
                    spos_v[f, :] = pos_v[pl.ds(j0, _L)]
                    pltpu.async_copy(
                        rows_v, out_hbm.at[spos_v.at[f]], osem).wait()

                return vb, ordn

            lax.fori_loop(0, _BPW, body, (jnp.int32(-1), jnp.int32(-1)))

        gather_one(ci_hbm, cp_hbm, wc_hbm, outc_hbm)
        gather_one(xi_hbm, xp_hbm, wx_hbm, outx_hbm)

    return k(ci_sorted.reshape(_NW, _BPW), ci_pos.reshape(_NW, _BPW),
             xi_sorted.reshape(_NW, _BPW), xi_pos.reshape(_NW, _BPW),
             Wc_t, Wx_t)


_FBLK = 2048


def _tc_finish_body(c_ref, x_ref, out_ref):
    d = lax.broadcasted_iota(jnp.int32, (1, _DPAD), 1)
    mask = (d < DIM).astype(jnp.float32)
    c = c_ref[...] * mask
    x = x_ref[...] * mask
    dot = jnp.sum(c * x, axis=1)
    c2 = jnp.sum(c * c, axis=1)
    x2 = jnp.sum(x * x, axis=1)
    scale_c = jnp.minimum(1.0, MAX_NORM / jnp.maximum(jnp.sqrt(c2), 1e-7))
    scale_x = jnp.minimum(1.0, MAX_NORM / jnp.maximum(jnp.sqrt(x2), 1e-7))
    s = dot * scale_c * scale_x
    loss = jax.nn.log_sigmoid(s)
    part = jnp.full((1, 1), -jnp.sum(loss) / BATCH, jnp.float32)

    @pl.when(pl.program_id(0) == 0)
    def _():
        out_ref[...] = jnp.zeros((1, 1), jnp.float32)

    out_ref[...] += part


def kernel(center_input, context_input, W_center, W_context):
    ci = center_input.astype(jnp.int32)
    xi = context_input.astype(jnp.int32)
    iota = lax.iota(jnp.int32, BATCH)
    ci_s, ci_pos = lax.sort((ci, iota), num_keys=1)
    xi_s, xi_pos = lax.sort((xi, iota), num_keys=1)
    rows_c, rows_x = _sc_gather2(
        ci_s, ci_pos, xi_s, xi_pos, W_center.T, W_context.T)
    res = pl.pallas_call(
        _tc_finish_body,
        grid=(BATCH // _FBLK,),
        in_specs=[
            pl.BlockSpec((_FBLK, _DPAD), lambda i: (i, 0)),
            pl.BlockSpec((_FBLK, _DPAD), lambda i: (i, 0)),
        ],
        out_specs=pl.BlockSpec((1, 1), lambda i: (0, 0)),
        out_shape=jax.ShapeDtypeStruct((1, 1), jnp.float32),
    )(rows_c, rows_x)
    return res[0, 0]
